# Initial kernel scaffold; baseline (speedup 1.0000x reference)
import jax
import jax.numpy as jnp
from jax.experimental import pallas as pl


def kernel(x, edge_index, syll_nodes, emb_table, W1, b1, W3, b3, pad_vector):
    raise NotImplementedError("write your pallas kernel here")


# speedup vs baseline: 17.2112x; 17.2112x over previous
import functools

import jax
import jax.numpy as jnp
from jax import lax
from jax.experimental import pallas as pl
from jax.experimental.pallas import tpu as pltpu
from jax.experimental.pallas import tpu_sc as plsc

N_NODES = 10000
N_PAD = 10240
N_EDGES = 320000
D = 128
NC, NS = 2, 16
EC = 125
ECR = 80
XC = 40
XCR = 8
STRIPE = N_PAD // NS

_MESH = plsc.VectorSubcoreMesh(
    core_axis_name="c", subcore_axis_name="s", num_cores=NC, num_subcores=NS)


@functools.partial(
    pl.kernel,
    out_type=(
        jax.ShapeDtypeStruct((N_PAD, D), jnp.float32),
        jax.ShapeDtypeStruct((NC * N_PAD,), jnp.float32),
    ),
    mesh=_MESH,
    scratch_types=[
        pltpu.VMEM((XCR, XC), jnp.int32),
        pltpu.VMEM((ECR, EC), jnp.int32),
        pltpu.VMEM((XC, D), jnp.float32),
        pltpu.VMEM((EC,), jnp.float32),
        pltpu.SemaphoreType.DMA,
        pltpu.VMEM_SHARED((N_PAD,), jnp.float32),
    ],
)
def _sc_emb_deg(x2d, dst2d, zeros1d, ones1d, emb_table, h0_out, deg_out,
                xidx_v, dstidx_v, rows_v, ones_v, sem, deg_sh):
    c = lax.axis_index("c")
    s = lax.axis_index("s")
    w = c * NS + s

    pltpu.sync_copy(x2d.at[pl.ds(w * XCR, XCR)], xidx_v)
    pltpu.sync_copy(dst2d.at[pl.ds(c * 1280 + s * ECR, ECR)], dstidx_v)
    pltpu.sync_copy(zeros1d, deg_sh.at[pl.ds(s * STRIPE, STRIPE)])
    pltpu.sync_copy(ones1d, ones_v)
    plsc.subcore_barrier()

    for j in range(XCR):
        pltpu.async_copy(emb_table.at[xidx_v.at[j]], rows_v, sem).wait()
        pltpu.sync_copy(rows_v, h0_out.at[pl.ds(w * (XCR * XC) + j * XC, XC)])

    def deg_body(i, carry):
        pltpu.sync_copy(ones_v, deg_sh.at[dstidx_v.at[i]], add=True)
        return carry
    lax.fori_loop(0, ECR, deg_body, 0)

    plsc.subcore_barrier()
    pltpu.sync_copy(deg_sh.at[pl.ds(s * STRIPE, STRIPE)],
                    deg_out.at[pl.ds(c * N_PAD + s * STRIPE, STRIPE)])


@functools.partial(
    pl.kernel,
    out_type=jax.ShapeDtypeStruct((NC, N_PAD, D), jnp.float32),
    mesh=_MESH,
    scratch_types=[
        pltpu.VMEM((ECR, EC), jnp.int32),
        pltpu.VMEM((ECR, EC), jnp.int32),
        pltpu.VMEM((EC, D), jnp.float32),
        pltpu.SemaphoreType.DMA,
        pltpu.VMEM_SHARED((N_PAD, D), jnp.float32),
    ],
)
def _sc_agg(hwp, src2d, dst2d, zeros2d, agg_out,
            srcidx_v, dstidx_v, rows_v, sem, agg_sh):
    c = lax.axis_index("c")
    s = lax.axis_index("s")
    rb = c * 1280 + s * ECR

    pltpu.sync_copy(src2d.at[pl.ds(rb, ECR)], srcidx_v)
    pltpu.sync_copy(dst2d.at[pl.ds(rb, ECR)], dstidx_v)
    pltpu.sync_copy(zeros2d, agg_sh.at[pl.ds(s * STRIPE, STRIPE)])
    plsc.subcore_barrier()

    def body(i, carry):
        pltpu.async_copy(hwp.at[srcidx_v.at[i]], rows_v, sem).wait()
        pltpu.sync_copy(rows_v, agg_sh.at[dstidx_v.at[i]], add=True)
        return carry
    lax.fori_loop(0, ECR, body, 0)

    plsc.subcore_barrier()
    pltpu.sync_copy(agg_sh.at[pl.ds(s * STRIPE, STRIPE)],
                    agg_out.at[c, pl.ds(s * STRIPE, STRIPE)])


@functools.partial(
    pl.kernel,
    out_type=jax.ShapeDtypeStruct((2048, D), jnp.float32),
    mesh=_MESH,
    scratch_types=[
        pltpu.VMEM((32, 64), jnp.int32),
        pltpu.VMEM((64, D), jnp.float32),
        pltpu.SemaphoreType.DMA,
    ],
)
def _sc_syll(h2, syll2d, g_out, idx_v, rows_v, sem):
    w = lax.axis_index("c") * NS + lax.axis_index("s")
    pltpu.sync_copy(syll2d, idx_v)
    pltpu.async_copy(h2.at[idx_v.at[w]], rows_v, sem).wait()
    pltpu.sync_copy(rows_v, g_out.at[pl.ds(w * 64, 64)])


def _tc_first_body(h0, d0, d1, W1, hw_out, dis_out):
    dis = lax.rsqrt(d0[...] + d1[...] + 1.0)
    dis_out[...] = dis
    hw_out[...] = dis * jnp.dot(h0[...], W1[...],
                                preferred_element_type=jnp.float32)


def _tc_first(h0, d0, d1, W1):
    return pl.pallas_call(
        _tc_first_body,
        out_shape=(
            jax.ShapeDtypeStruct((N_PAD, D), jnp.float32),
            jax.ShapeDtypeStruct((N_PAD, 1), jnp.float32),
        ),
    )(h0, d0, d1, W1)


def _tc_mid_body(a0, a1, hwp, dis, b1, W3, out):
    h1 = jnp.maximum(dis[...] * (a0[...] + a1[...] + hwp[...]) + b1[...], 0.0)
    out[...] = dis[...] * jnp.dot(h1, W3[...],
                                  preferred_element_type=jnp.float32)


def _tc_mid(a0, a1, hwp, dis, b1, W3):
    return pl.pallas_call(
        _tc_mid_body,
        out_shape=jax.ShapeDtypeStruct((N_PAD, D), jnp.float32),
    )(a0, a1, hwp, dis, b1, W3)


def _tc_last_body(a0, a1, hwp, dis, b3, out):
    out[...] = dis[...] * (a0[...] + a1[...] + hwp[...]) + b3[...]


def _tc_last(a0, a1, hwp, dis, b3):
    return pl.pallas_call(
        _tc_last_body,
        out_shape=jax.ShapeDtypeStruct((N_PAD, D), jnp.float32),
    )(a0, a1, hwp, dis, b3)


def kernel(x, edge_index, syll_nodes, emb_table, W1, b1, W3, b3, pad_vector):
    del pad_vector
    x_p = jnp.pad(x.astype(jnp.int32), (0, N_PAD - N_NODES)).reshape(256, XC)
    src2 = edge_index[0].astype(jnp.int32).reshape(N_EDGES // EC, EC)
    dst2 = edge_index[1].astype(jnp.int32).reshape(N_EDGES // EC, EC)
    syll2 = syll_nodes.astype(jnp.int32).reshape(32, 64)
    zeros1 = jnp.zeros((STRIPE,), jnp.float32)
    ones1 = jnp.ones((EC,), jnp.float32)
    zeros2 = jnp.zeros((STRIPE, D), jnp.float32)

    h0, degf = _sc_emb_deg(x_p, dst2, zeros1, ones1, emb_table)
    degp = degf.reshape(NC, N_PAD)
    hw1p, dis = _tc_first(h0, degp[0][:, None], degp[1][:, None], W1)
    agg1 = _sc_agg(hw1p, src2, dst2, zeros2)
    hw3p = _tc_mid(agg1[0], agg1[1], hw1p, dis, b1.reshape(1, D), W3)
    agg3 = _sc_agg(hw3p, src2, dst2, zeros2)
    h2 = _tc_last(agg3[0], agg3[1], hw3p, dis, b3.reshape(1, D))
    g = _sc_syll(h2, syll2)
    return g[None]


# speedup vs baseline: 23.6405x; 1.3736x over previous
import functools

import jax
import jax.numpy as jnp
from jax import lax
from jax.experimental import pallas as pl
from jax.experimental.pallas import tpu as pltpu
from jax.experimental.pallas import tpu_sc as plsc

N_NODES = 10000
N_PAD = 10240
N_EDGES = 320000
D = 128
NC, NS = 2, 16
EC = 125
ECR = 80
ECRH = 40
XC = 40
XCR = 8
STRIPE = N_PAD // NS

_MESH = plsc.VectorSubcoreMesh(
    core_axis_name="c", subcore_axis_name="s", num_cores=NC, num_subcores=NS)


@functools.partial(
    pl.kernel,
    out_type=(
        jax.ShapeDtypeStruct((N_PAD, D), jnp.float32),
        jax.ShapeDtypeStruct((NC * N_PAD,), jnp.float32),
    ),
    mesh=_MESH,
    scratch_types=[
        pltpu.VMEM((XCR, XC), jnp.int32),
        pltpu.VMEM((ECR, EC), jnp.int32),
        pltpu.VMEM((XC, D), jnp.float32),
        pltpu.VMEM((EC,), jnp.float32),
        pltpu.SemaphoreType.DMA,
        pltpu.VMEM_SHARED((N_PAD,), jnp.float32),
    ],
)
def _sc_emb_deg(x2d, dst2d, zeros1d, ones1d, emb_table, h0_out, deg_out,
                xidx_v, dstidx_v, rows_v, ones_v, sem, deg_sh):
    c = lax.axis_index("c")
    s = lax.axis_index("s")
    w = c * NS + s

    pltpu.sync_copy(x2d.at[pl.ds(w * XCR, XCR)], xidx_v)
    pltpu.sync_copy(dst2d.at[pl.ds(c * 1280 + s * ECR, ECR)], dstidx_v)
    pltpu.sync_copy(zeros1d, deg_sh.at[pl.ds(s * STRIPE, STRIPE)])
    pltpu.sync_copy(ones1d, ones_v)
    plsc.subcore_barrier()

    for j in range(XCR):
        pltpu.async_copy(emb_table.at[xidx_v.at[j]], rows_v, sem).wait()
        pltpu.sync_copy(rows_v, h0_out.at[pl.ds(w * (XCR * XC) + j * XC, XC)])

    def deg_body(i, carry):
        pltpu.sync_copy(ones_v, deg_sh.at[dstidx_v.at[i]], add=True)
        return carry
    lax.fori_loop(0, ECR, deg_body, 0)

    plsc.subcore_barrier()
    pltpu.sync_copy(deg_sh.at[pl.ds(s * STRIPE, STRIPE)],
                    deg_out.at[pl.ds(c * N_PAD + s * STRIPE, STRIPE)])


@functools.partial(
    pl.kernel,
    out_type=jax.ShapeDtypeStruct((NC, N_PAD, D), jnp.float32),
    mesh=_MESH,
    scratch_types=[
        pltpu.VMEM((ECRH, EC), jnp.int32),
        pltpu.VMEM((ECRH, EC), jnp.int32),
        pltpu.VMEM((2, EC, D), jnp.float32),
        pltpu.SemaphoreType.DMA((2,)),
        pltpu.VMEM_SHARED((N_PAD, D), jnp.float32),
    ],
)
def _sc_agg(hwp, src2d, dst2d, zeros2d, agg_out,
            srcidx_v, dstidx_v, rows_v, sem, agg_sh):
    c = lax.axis_index("c")
    s = lax.axis_index("s")
    rb = c * 1280 + s * ECR

    pltpu.sync_copy(zeros2d, agg_sh.at[pl.ds(s * STRIPE, STRIPE)])
    plsc.subcore_barrier()

    for half in range(ECR // ECRH):
        pltpu.sync_copy(src2d.at[pl.ds(rb + half * ECRH, ECRH)], srcidx_v)
        pltpu.sync_copy(dst2d.at[pl.ds(rb + half * ECRH, ECRH)], dstidx_v)
        pltpu.async_copy(hwp.at[srcidx_v.at[0]], rows_v.at[0], sem.at[0])

        def body(i, carry):
            nxt = i + 1

            @pl.when(nxt < ECRH)
            def _():
                pltpu.async_copy(hwp.at[srcidx_v.at[nxt]], rows_v.at[nxt % 2],
                                 sem.at[nxt % 2])

            pltpu.make_async_copy(hwp.at[srcidx_v.at[i]], rows_v.at[i % 2],
                                  sem.at[i % 2]).wait()
            pltpu.sync_copy(rows_v.at[i % 2], agg_sh.at[dstidx_v.at[i]],
                            add=True)
            return carry
        lax.fori_loop(0, ECRH, body, 0)

    plsc.subcore_barrier()
    pltpu.sync_copy(agg_sh.at[pl.ds(s * STRIPE, STRIPE)],
                    agg_out.at[c, pl.ds(s * STRIPE, STRIPE)])


@functools.partial(
    pl.kernel,
    out_type=jax.ShapeDtypeStruct((2048, D), jnp.float32),
    mesh=_MESH,
    scratch_types=[
        pltpu.VMEM((32, 64), jnp.int32),
        pltpu.VMEM((64, D), jnp.float32),
        pltpu.SemaphoreType.DMA,
    ],
)
def _sc_syll(h2, syll2d, g_out, idx_v, rows_v, sem):
    w = lax.axis_index("c") * NS + lax.axis_index("s")
    pltpu.sync_copy(syll2d, idx_v)
    pltpu.async_copy(h2.at[idx_v.at[w]], rows_v, sem).wait()
    pltpu.sync_copy(rows_v, g_out.at[pl.ds(w * 64, 64)])


def _tc_first_body(h0, d0, d1, W1, hw_out, dis_out):
    dis = lax.rsqrt(d0[...] + d1[...] + 1.0)
    dis_out[...] = dis
    hw_out[...] = dis * jnp.dot(h0[...], W1[...],
                                preferred_element_type=jnp.float32)


def _tc_first(h0, d0, d1, W1):
    return pl.pallas_call(
        _tc_first_body,
        out_shape=(
            jax.ShapeDtypeStruct((N_PAD, D), jnp.float32),
            jax.ShapeDtypeStruct((N_PAD, 1), jnp.float32),
        ),
    )(h0, d0, d1, W1)


def _tc_mid_body(a0, a1, hwp, dis, b1, W3, out):
    h1 = jnp.maximum(dis[...] * (a0[...] + a1[...] + hwp[...]) + b1[...], 0.0)
    out[...] = dis[...] * jnp.dot(h1, W3[...],
                                  preferred_element_type=jnp.float32)


def _tc_mid(a0, a1, hwp, dis, b1, W3):
    return pl.pallas_call(
        _tc_mid_body,
        out_shape=jax.ShapeDtypeStruct((N_PAD, D), jnp.float32),
    )(a0, a1, hwp, dis, b1, W3)


def _tc_last_body(a0, a1, hwp, dis, b3, out):
    out[...] = dis[...] * (a0[...] + a1[...] + hwp[...]) + b3[...]


def _tc_last(a0, a1, hwp, dis, b3):
    return pl.pallas_call(
        _tc_last_body,
        out_shape=jax.ShapeDtypeStruct((N_PAD, D), jnp.float32),
    )(a0, a1, hwp, dis, b3)


def kernel(x, edge_index, syll_nodes, emb_table, W1, b1, W3, b3, pad_vector):
    del pad_vector
    x_p = jnp.pad(x.astype(jnp.int32), (0, N_PAD - N_NODES)).reshape(256, XC)
    src2 = edge_index[0].astype(jnp.int32).reshape(N_EDGES // EC, EC)
    dst2 = edge_index[1].astype(jnp.int32).reshape(N_EDGES // EC, EC)
    syll2 = syll_nodes.astype(jnp.int32).reshape(32, 64)
    zeros1 = jnp.zeros((STRIPE,), jnp.float32)
    ones1 = jnp.ones((EC,), jnp.float32)
    zeros2 = jnp.zeros((STRIPE, D), jnp.float32)

    h0, degf = _sc_emb_deg(x_p, dst2, zeros1, ones1, emb_table)
    degp = degf.reshape(NC, N_PAD)
    hw1p, dis = _tc_first(h0, degp[0][:, None], degp[1][:, None], W1)
    agg1 = _sc_agg(hw1p, src2, dst2, zeros2)
    hw3p = _tc_mid(agg1[0], agg1[1], hw1p, dis, b1.reshape(1, D), W3)
    agg3 = _sc_agg(hw3p, src2, dst2, zeros2)
    h2 = _tc_last(agg3[0], agg3[1], hw3p, dis, b3.reshape(1, D))
    g = _sc_syll(h2, syll2)
    return g[None]


# speedup vs baseline: 24.2639x; 1.0264x over previous
import functools

import jax
import jax.numpy as jnp
from jax import lax
from jax.experimental import pallas as pl
from jax.experimental.pallas import tpu as pltpu
from jax.experimental.pallas import tpu_sc as plsc

N_NODES = 10000
N_PAD = 10240
N_EDGES = 320000
D = 128
NC, NS = 2, 16
EC = 125
ECR = 80
ECRH = 40
XC = 40
XCR = 8
STRIPE = N_PAD // NS

_MESH = plsc.VectorSubcoreMesh(
    core_axis_name="c", subcore_axis_name="s", num_cores=NC, num_subcores=NS)


@functools.partial(
    pl.kernel,
    out_type=(
        jax.ShapeDtypeStruct((N_PAD, D), jnp.float32),
        jax.ShapeDtypeStruct((NC * N_PAD,), jnp.float32),
    ),
    mesh=_MESH,
    scratch_types=[
        pltpu.VMEM((XCR, XC), jnp.int32),
        pltpu.VMEM((ECR, EC), jnp.int32),
        pltpu.VMEM((2, XC, D), jnp.float32),
        pltpu.VMEM((EC,), jnp.float32),
        pltpu.SemaphoreType.DMA((2,)),
        pltpu.SemaphoreType.DMA,
        pltpu.VMEM_SHARED((N_PAD,), jnp.float32),
    ],
)
def _sc_emb_deg(x2d, dst2d, zeros1d, ones1d, emb_table, h0_out, deg_out,
                xidx_v, dstidx_v, rows_v, ones_v, semg, semd, deg_sh):
    c = lax.axis_index("c")
    s = lax.axis_index("s")
    w = c * NS + s

    pltpu.sync_copy(x2d.at[pl.ds(w * XCR, XCR)], xidx_v)
    pltpu.sync_copy(dst2d.at[pl.ds(c * 1280 + s * ECR, ECR)], dstidx_v)
    pltpu.sync_copy(zeros1d, deg_sh.at[pl.ds(s * STRIPE, STRIPE)])
    pltpu.sync_copy(ones1d, ones_v)
    plsc.subcore_barrier()

    def deg_issue(i, carry):
        pltpu.async_copy(ones_v, deg_sh.at[dstidx_v.at[i]], semd, add=True)
        return carry
    lax.fori_loop(0, ECR, deg_issue, 0)

    pltpu.async_copy(emb_table.at[xidx_v.at[0]], rows_v.at[0], semg.at[0])
    for j in range(XCR):
        if j + 1 < XCR:
            pltpu.async_copy(emb_table.at[xidx_v.at[j + 1]],
                             rows_v.at[(j + 1) % 2], semg.at[(j + 1) % 2])
        pltpu.make_async_copy(emb_table.at[xidx_v.at[j]], rows_v.at[j % 2],
                              semg.at[j % 2]).wait()
        pltpu.sync_copy(rows_v.at[j % 2],
                        h0_out.at[pl.ds(w * (XCR * XC) + j * XC, XC)])

    def deg_drain(i, carry):
        pltpu.make_async_copy(ones_v, deg_sh.at[dstidx_v.at[i]], semd).wait()
        return carry
    lax.fori_loop(0, ECR, deg_drain, 0)

    plsc.subcore_barrier()
    pltpu.sync_copy(deg_sh.at[pl.ds(s * STRIPE, STRIPE)],
                    deg_out.at[pl.ds(c * N_PAD + s * STRIPE, STRIPE)])


@functools.partial(
    pl.kernel,
    out_type=jax.ShapeDtypeStruct((NC, N_PAD, D), jnp.float32),
    mesh=_MESH,
    scratch_types=[
        pltpu.VMEM((ECRH, EC), jnp.int32),
        pltpu.VMEM((ECRH, EC), jnp.int32),
        pltpu.VMEM((2, EC, D), jnp.float32),
        pltpu.SemaphoreType.DMA((2,)),
        pltpu.VMEM_SHARED((N_PAD, D), jnp.float32),
    ],
)
def _sc_agg(hwp, src2d, dst2d, zeros2d, agg_out,
            srcidx_v, dstidx_v, rows_v, sem, agg_sh):
    c = lax.axis_index("c")
    s = lax.axis_index("s")
    rb = c * 1280 + s * ECR

    pltpu.sync_copy(zeros2d, agg_sh.at[pl.ds(s * STRIPE, STRIPE)])
    plsc.subcore_barrier()

    for half in range(ECR // ECRH):
        pltpu.sync_copy(src2d.at[pl.ds(rb + half * ECRH, ECRH)], srcidx_v)
        pltpu.sync_copy(dst2d.at[pl.ds(rb + half * ECRH, ECRH)], dstidx_v)
        pltpu.async_copy(hwp.at[srcidx_v.at[0]], rows_v.at[0], sem.at[0])

        def body(i, carry):
            nxt = i + 1

            @pl.when(nxt < ECRH)
            def _():
                pltpu.async_copy(hwp.at[srcidx_v.at[nxt]], rows_v.at[nxt % 2],
                                 sem.at[nxt % 2])

            pltpu.make_async_copy(hwp.at[srcidx_v.at[i]], rows_v.at[i % 2],
                                  sem.at[i % 2]).wait()
            pltpu.sync_copy(rows_v.at[i % 2], agg_sh.at[dstidx_v.at[i]],
                            add=True)
            return carry
        lax.fori_loop(0, ECRH, body, 0)

    plsc.subcore_barrier()
    pltpu.sync_copy(agg_sh.at[pl.ds(s * STRIPE, STRIPE)],
                    agg_out.at[c, pl.ds(s * STRIPE, STRIPE)])


@functools.partial(
    pl.kernel,
    out_type=jax.ShapeDtypeStruct((2048, D), jnp.float32),
    mesh=_MESH,
    scratch_types=[
        pltpu.VMEM((32, 64), jnp.int32),
        pltpu.VMEM((64, D), jnp.float32),
        pltpu.VMEM((64, D), jnp.float32),
        pltpu.VMEM((64, D), jnp.float32),
        pltpu.VMEM((64, D), jnp.float32),
        pltpu.VMEM((64,), jnp.float32),
        pltpu.VMEM((D,), jnp.float32),
        pltpu.SemaphoreType.DMA((4,)),
    ],
)
def _sc_final(a0, a1, hwp3, dis1, b3r, syll2d, g_out,
              sidx_v, r0, r1, r2, ro, dv, b3_v, sem):
    w = lax.axis_index("c") * NS + lax.axis_index("s")
    pltpu.sync_copy(syll2d, sidx_v)
    pltpu.sync_copy(b3r, b3_v)
    idxrow = sidx_v.at[w]
    cps = [
        pltpu.async_copy(a0.at[idxrow], r0, sem.at[0]),
        pltpu.async_copy(a1.at[idxrow], r1, sem.at[1]),
        pltpu.async_copy(hwp3.at[idxrow], r2, sem.at[2]),
        pltpu.async_copy(dis1.at[idxrow], dv, sem.at[3]),
    ]
    for cp in cps:
        cp.wait()

    for r in range(64):
        if r % 16 == 0:
            dvec = dv[pl.ds(r, 16)]
        d = dvec[r % 16]
        for k in range(D // 16):
            sl = pl.ds(k * 16, 16)
            ro[r, sl] = d * (r0[r, sl] + r1[r, sl] + r2[r, sl]) + b3_v[sl]

    pltpu.sync_copy(ro, g_out.at[pl.ds(w * 64, 64)])


def _tc_first_body(h0, d0, d1, W1, hw_out, dis_out):
    dis = lax.rsqrt(d0[...] + d1[...] + 1.0)
    dis_out[...] = dis
    hw_out[...] = dis * jnp.dot(h0[...], W1[...],
                                preferred_element_type=jnp.float32)


def _tc_first(h0, d0, d1, W1):
    return pl.pallas_call(
        _tc_first_body,
        out_shape=(
            jax.ShapeDtypeStruct((N_PAD, D), jnp.float32),
            jax.ShapeDtypeStruct((N_PAD, 1), jnp.float32),
        ),
    )(h0, d0, d1, W1)


def _tc_mid_body(a0, a1, hwp, dis, b1, W3, out):
    h1 = jnp.maximum(dis[...] * (a0[...] + a1[...] + hwp[...]) + b1[...], 0.0)
    out[...] = dis[...] * jnp.dot(h1, W3[...],
                                  preferred_element_type=jnp.float32)


def _tc_mid(a0, a1, hwp, dis, b1, W3):
    return pl.pallas_call(
        _tc_mid_body,
        out_shape=jax.ShapeDtypeStruct((N_PAD, D), jnp.float32),
    )(a0, a1, hwp, dis, b1, W3)


def kernel(x, edge_index, syll_nodes, emb_table, W1, b1, W3, b3, pad_vector):
    del pad_vector
    x_p = jnp.pad(x.astype(jnp.int32), (0, N_PAD - N_NODES)).reshape(256, XC)
    src2 = edge_index[0].astype(jnp.int32).reshape(N_EDGES // EC, EC)
    dst2 = edge_index[1].astype(jnp.int32).reshape(N_EDGES // EC, EC)
    syll2 = syll_nodes.astype(jnp.int32).reshape(32, 64)
    zeros1 = jnp.zeros((STRIPE,), jnp.float32)
    ones1 = jnp.ones((EC,), jnp.float32)
    zeros2 = jnp.zeros((STRIPE, D), jnp.float32)

    h0, degf = _sc_emb_deg(x_p, dst2, zeros1, ones1, emb_table)
    degp = degf.reshape(NC, N_PAD)
    hw1p, dis = _tc_first(h0, degp[0][:, None], degp[1][:, None], W1)
    agg1 = _sc_agg(hw1p, src2, dst2, zeros2)
    hw3p = _tc_mid(agg1[0], agg1[1], hw1p, dis, b1.reshape(1, D), W3)
    agg3 = _sc_agg(hw3p, src2, dst2, zeros2)
    g = _sc_final(agg3[0], agg3[1], hw3p, dis.reshape(N_PAD), b3, syll2)
    return g[None]


# speedup vs baseline: 24.6062x; 1.0141x over previous
import functools

import jax
import jax.numpy as jnp
from jax import lax
from jax.experimental import pallas as pl
from jax.experimental.pallas import tpu as pltpu
from jax.experimental.pallas import tpu_sc as plsc

N_NODES = 10000
N_PAD = 10240
N_EDGES = 320000
D = 128
NC, NS = 2, 16
EC = 125
ECR = 80
ECRH = 40
XC = 40
XCR = 8
STRIPE = N_PAD // NS

_MESH = plsc.VectorSubcoreMesh(
    core_axis_name="c", subcore_axis_name="s", num_cores=NC, num_subcores=NS)


@functools.partial(
    pl.kernel,
    out_type=(
        jax.ShapeDtypeStruct((N_PAD, D), jnp.float32),
        jax.ShapeDtypeStruct((NC * N_PAD,), jnp.float32),
    ),
    mesh=_MESH,
    scratch_types=[
        pltpu.VMEM((XCR, XC), jnp.int32),
        pltpu.VMEM((ECR, EC), jnp.int32),
        pltpu.VMEM((2, XC, D), jnp.float32),
        pltpu.VMEM((EC,), jnp.float32),
        pltpu.SemaphoreType.DMA((2,)),
        pltpu.SemaphoreType.DMA,
        pltpu.VMEM_SHARED((N_PAD,), jnp.float32),
    ],
)
def _sc_emb_deg(x2d, dst2d, zeros1d, ones1d, emb_table, h0_out, deg_out,
                xidx_v, dstidx_v, rows_v, ones_v, semg, semd, deg_sh):
    c = lax.axis_index("c")
    s = lax.axis_index("s")
    w = c * NS + s

    pltpu.sync_copy(x2d.at[pl.ds(w * XCR, XCR)], xidx_v)
    pltpu.sync_copy(dst2d.at[pl.ds(c * 1280 + s * ECR, ECR)], dstidx_v)
    pltpu.sync_copy(zeros1d, deg_sh.at[pl.ds(s * STRIPE, STRIPE)])
    pltpu.sync_copy(ones1d, ones_v)
    plsc.subcore_barrier()

    def deg_issue(i, carry):
        pltpu.async_copy(ones_v, deg_sh.at[dstidx_v.at[i]], semd, add=True)
        return carry
    lax.fori_loop(0, ECR, deg_issue, 0)

    pltpu.async_copy(emb_table.at[xidx_v.at[0]], rows_v.at[0], semg.at[0])
    for j in range(XCR):
        if j + 1 < XCR:
            pltpu.async_copy(emb_table.at[xidx_v.at[j + 1]],
                             rows_v.at[(j + 1) % 2], semg.at[(j + 1) % 2])
        pltpu.make_async_copy(emb_table.at[xidx_v.at[j]], rows_v.at[j % 2],
                              semg.at[j % 2]).wait()
        pltpu.sync_copy(rows_v.at[j % 2],
                        h0_out.at[pl.ds(w * (XCR * XC) + j * XC, XC)])

    def deg_drain(i, carry):
        pltpu.make_async_copy(ones_v, deg_sh.at[dstidx_v.at[i]], semd).wait()
        return carry
    lax.fori_loop(0, ECR, deg_drain, 0)

    plsc.subcore_barrier()
    pltpu.sync_copy(deg_sh.at[pl.ds(s * STRIPE, STRIPE)],
                    deg_out.at[pl.ds(c * N_PAD + s * STRIPE, STRIPE)])


EC2 = 50
AGR = N_EDGES // EC2
APT = AGR // NC // NS
GSZ = 40


@functools.partial(
    pl.kernel,
    out_type=jax.ShapeDtypeStruct((NC, N_PAD, D), jnp.float32),
    mesh=_MESH,
    scratch_types=[
        pltpu.VMEM((GSZ, EC2), jnp.int32),
        pltpu.VMEM((GSZ, EC2), jnp.int32),
        pltpu.VMEM((4, EC2, D), jnp.float32),
        pltpu.SemaphoreType.DMA((4,)),
        pltpu.VMEM_SHARED((N_PAD, D), jnp.float32),
    ],
)
def _sc_agg(hwp, src2d, dst2d, zeros2d, agg_out,
            srcidx_v, dstidx_v, rows_v, sem, agg_sh):
    c = lax.axis_index("c")
    s = lax.axis_index("s")
    rb = (c * NS + s) * APT

    pltpu.sync_copy(zeros2d, agg_sh.at[pl.ds(s * STRIPE, STRIPE)])
    plsc.subcore_barrier()

    for g in range(APT // GSZ):
        pltpu.sync_copy(src2d.at[pl.ds(rb + g * GSZ, GSZ)], srcidx_v)
        pltpu.sync_copy(dst2d.at[pl.ds(rb + g * GSZ, GSZ)], dstidx_v)
        for p in range(3):
            pltpu.async_copy(hwp.at[srcidx_v.at[p]], rows_v.at[p], sem.at[p])

        def body(i, carry):
            nxt = i + 3

            @pl.when(nxt < GSZ)
            def _():
                pltpu.async_copy(hwp.at[srcidx_v.at[nxt]], rows_v.at[nxt % 4],
                                 sem.at[nxt % 4])

            pltpu.make_async_copy(hwp.at[srcidx_v.at[i]], rows_v.at[i % 4],
                                  sem.at[i % 4]).wait()
            pltpu.sync_copy(rows_v.at[i % 4], agg_sh.at[dstidx_v.at[i]],
                            add=True)
            return carry
        lax.fori_loop(0, GSZ, body, 0)

    plsc.subcore_barrier()
    pltpu.sync_copy(agg_sh.at[pl.ds(s * STRIPE, STRIPE)],
                    agg_out.at[c, pl.ds(s * STRIPE, STRIPE)])


@functools.partial(
    pl.kernel,
    out_type=jax.ShapeDtypeStruct((2048, D), jnp.float32),
    mesh=_MESH,
    scratch_types=[
        pltpu.VMEM((32, 64), jnp.int32),
        pltpu.VMEM((64, D), jnp.float32),
        pltpu.VMEM((64, D), jnp.float32),
        pltpu.VMEM((64, D), jnp.float32),
        pltpu.VMEM((64, D), jnp.float32),
        pltpu.VMEM((64,), jnp.float32),
        pltpu.VMEM((D,), jnp.float32),
        pltpu.SemaphoreType.DMA((4,)),
    ],
)
def _sc_final(a0, a1, hwp3, dis1, b3r, syll2d, g_out,
              sidx_v, r0, r1, r2, ro, dv, b3_v, sem):
    w = lax.axis_index("c") * NS + lax.axis_index("s")
    pltpu.sync_copy(syll2d, sidx_v)
    pltpu.sync_copy(b3r, b3_v)
    idxrow = sidx_v.at[w]
    cps = [
        pltpu.async_copy(a0.at[idxrow], r0, sem.at[0]),
        pltpu.async_copy(a1.at[idxrow], r1, sem.at[1]),
        pltpu.async_copy(hwp3.at[idxrow], r2, sem.at[2]),
        pltpu.async_copy(dis1.at[idxrow], dv, sem.at[3]),
    ]
    for cp in cps:
        cp.wait()

    for r in range(64):
        if r % 16 == 0:
            dvec = dv[pl.ds(r, 16)]
        d = dvec[r % 16]
        for k in range(D // 16):
            sl = pl.ds(k * 16, 16)
            ro[r, sl] = d * (r0[r, sl] + r1[r, sl] + r2[r, sl]) + b3_v[sl]

    pltpu.sync_copy(ro, g_out.at[pl.ds(w * 64, 64)])


def _tc_first_body(h0, d0, d1, W1, hw_out, dis_out):
    dis = lax.rsqrt(d0[...] + d1[...] + 1.0)
    dis_out[...] = dis
    hw_out[...] = dis * jnp.dot(h0[...], W1[...],
                                preferred_element_type=jnp.float32)


def _tc_first(h0, d0, d1, W1):
    return pl.pallas_call(
        _tc_first_body,
        out_shape=(
            jax.ShapeDtypeStruct((N_PAD, D), jnp.float32),
            jax.ShapeDtypeStruct((N_PAD, 1), jnp.float32),
        ),
    )(h0, d0, d1, W1)


def _tc_mid_body(a0, a1, hwp, dis, b1, W3, out):
    h1 = jnp.maximum(dis[...] * (a0[...] + a1[...] + hwp[...]) + b1[...], 0.0)
    out[...] = dis[...] * jnp.dot(h1, W3[...],
                                  preferred_element_type=jnp.float32)


def _tc_mid(a0, a1, hwp, dis, b1, W3):
    return pl.pallas_call(
        _tc_mid_body,
        out_shape=jax.ShapeDtypeStruct((N_PAD, D), jnp.float32),
    )(a0, a1, hwp, dis, b1, W3)


def kernel(x, edge_index, syll_nodes, emb_table, W1, b1, W3, b3, pad_vector):
    del pad_vector
    x_p = jnp.pad(x.astype(jnp.int32), (0, N_PAD - N_NODES)).reshape(256, XC)
    src2 = edge_index[0].astype(jnp.int32).reshape(AGR, EC2)
    dst2 = edge_index[1].astype(jnp.int32).reshape(AGR, EC2)
    dst2k1 = edge_index[1].astype(jnp.int32).reshape(N_EDGES // EC, EC)
    syll2 = syll_nodes.astype(jnp.int32).reshape(32, 64)
    zeros1 = jnp.zeros((STRIPE,), jnp.float32)
    ones1 = jnp.ones((EC,), jnp.float32)
    zeros2 = jnp.zeros((STRIPE, D), jnp.float32)

    h0, degf = _sc_emb_deg(x_p, dst2k1, zeros1, ones1, emb_table)
    degp = degf.reshape(NC, N_PAD)
    hw1p, dis = _tc_first(h0, degp[0][:, None], degp[1][:, None], W1)
    agg1 = _sc_agg(hw1p, src2, dst2, zeros2)
    hw3p = _tc_mid(agg1[0], agg1[1], hw1p, dis, b1.reshape(1, D), W3)
    agg3 = _sc_agg(hw3p, src2, dst2, zeros2)
    g = _sc_final(agg3[0], agg3[1], hw3p, dis.reshape(N_PAD), b3, syll2)
    return g[None]
